# Initial kernel scaffold; baseline (speedup 1.0000x reference)
import jax
import jax.numpy as jnp
from jax.experimental import pallas as pl


def kernel(node_feats, node_attrs, edge_feats, edge_attrs, W_up, W_r1, W_r2, W_out0, W_out1, W_sc, edge_index):
    raise NotImplementedError("write your pallas kernel here")


# speedup vs baseline: 2.5978x; 2.5978x over previous
import functools
import math

import jax
import jax.numpy as jnp
import numpy as np
from jax import lax
from jax.experimental import pallas as pl
from jax.experimental.pallas import tpu as pltpu
from jax.experimental.pallas import tpu_sc as plsc

_N = 10000
_E = 320000
_D = 128
_AVG_NEIGH = 32.0

_NC = 2
_NS = 16
_NW = _NC * _NS
_K = 128

_EROWS = 2528
_E_PAD = _EROWS * _K

_ROWS_PER_W = _EROWS // _NW
_ROWS_PER_T = _EROWS // _NS
_N_PAD = 10240
_NODES_PER_T = _N_PAD // _NS


def _tc_linear_up(node_feats, W_up):
    def body(nf_ref, w_ref, o_ref):
        o_ref[...] = jnp.dot(nf_ref[...], w_ref[...],
                             preferred_element_type=jnp.float32) * (1.0 / math.sqrt(_D))

    return pl.pallas_call(
        body,
        grid=(10,),
        in_specs=[
            pl.BlockSpec((1000, _D), lambda i: (i, 0)),
            pl.BlockSpec((_D, _D), lambda i: (0, 0)),
        ],
        out_specs=pl.BlockSpec((1000, _D), lambda i: (i, 0)),
        out_shape=jax.ShapeDtypeStruct((_N, _D), jnp.float32),
    )(node_feats, W_up)


def _sc_gather(x, src2):
    mesh = plsc.VectorSubcoreMesh(core_axis_name="c", subcore_axis_name="s",
                                  num_cores=_NC, num_subcores=_NS)

    @functools.partial(
        pl.kernel,
        mesh=mesh,
        out_type=jax.ShapeDtypeStruct((_E_PAD, _D), jnp.float32),
        scratch_types=[
            pltpu.VMEM((_K,), jnp.int32),
            pltpu.VMEM((_K, _D), jnp.float32),
            pltpu.SemaphoreType.DMA,
        ],
    )
    def k(x_hbm, src_hbm, out_hbm, idx_v, rows_v, sem):
        wid = lax.axis_index("s") * _NC + lax.axis_index("c")

        def body(b, carry):
            r = wid * _ROWS_PER_W + b
            pltpu.sync_copy(src_hbm.at[r], idx_v)
            pltpu.async_copy(x_hbm.at[idx_v], rows_v, sem).wait()
            pltpu.sync_copy(rows_v, out_hbm.at[pl.ds(r * _K, _K)])
            return carry

        lax.fori_loop(0, _ROWS_PER_W, body, 0)

    return k(x, src2)


def _tc_messages(edge_feats_p, edge_attrs_p, xs, W_r1, W_r2):
    def body(ef_ref, ea_ref, xs_ref, w1_ref, w2_ref, m_ref):
        h = jax.nn.silu(jnp.dot(ef_ref[...], w1_ref[...],
                                preferred_element_type=jnp.float32) * (1.0 / math.sqrt(8.0)))
        w = jnp.dot(h, w2_ref[...], preferred_element_type=jnp.float32) * (1.0 / math.sqrt(64.0))
        wA = w[:, :_D]
        wB = w[:, _D:]
        xs = xs_ref[...]
        scale = 1.0 / _AVG_NEIGH
        m_ref[0] = (wA * xs) * ea_ref[:, 0:1] * scale
        wx = wB * xs
        m_ref[1] = wx * ea_ref[:, 1:2] * scale
        m_ref[2] = wx * ea_ref[:, 2:3] * scale
        m_ref[3] = wx * ea_ref[:, 3:4] * scale

    _BE = 2048
    return pl.pallas_call(
        body,
        grid=(_E_PAD // _BE,),
        in_specs=[
            pl.BlockSpec((_BE, 8), lambda i: (i, 0)),
            pl.BlockSpec((_BE, 4), lambda i: (i, 0)),
            pl.BlockSpec((_BE, _D), lambda i: (i, 0)),
            pl.BlockSpec((8, 64), lambda i: (0, 0)),
            pl.BlockSpec((64, 2 * _D), lambda i: (0, 0)),
        ],
        out_specs=pl.BlockSpec((4, _BE, _D), lambda i: (0, i, 0)),
        out_shape=jax.ShapeDtypeStruct((4, _E_PAD, _D), jnp.float32),
    )(edge_feats_p, edge_attrs_p, xs, W_r1, W_r2)


def _sc_scatter(m, dst2, zeros_tile):
    mesh = plsc.VectorSubcoreMesh(core_axis_name="c", subcore_axis_name="s",
                                  num_cores=_NC, num_subcores=_NS)

    @functools.partial(
        pl.kernel,
        mesh=mesh,
        out_type=jax.ShapeDtypeStruct((4, _N_PAD, _D), jnp.float32),
        scratch_types=[
            pltpu.VMEM((_K,), jnp.int32),
            pltpu.VMEM((_K, _D), jnp.float32),
            pltpu.VMEM_SHARED((_N_PAD, _D), jnp.float32),
        ],
    )
    def k(m_hbm, dst_hbm, z_hbm, out_hbm, idx_v, mbuf, acc):
        cid = lax.axis_index("c")
        tid = lax.axis_index("s")
        nbase = tid * _NODES_PER_T
        nchunks = _NODES_PER_T // _K

        for gi in range(2):
            g = cid + 2 * gi
            pltpu.sync_copy(z_hbm, mbuf)
            for z in range(nchunks):
                pltpu.sync_copy(mbuf, acc.at[pl.ds(nbase + z * _K, _K)])
            plsc.subcore_barrier()

            def body(b, carry):
                r = tid * _ROWS_PER_T + b
                pltpu.sync_copy(dst_hbm.at[r], idx_v)
                pltpu.sync_copy(m_hbm.at[g, pl.ds(r * _K, _K)], mbuf)
                pltpu.sync_copy(mbuf, acc.at[idx_v], add=True)
                return carry

            lax.fori_loop(0, _ROWS_PER_T, body, 0)
            plsc.subcore_barrier()
            for z in range(nchunks):
                pltpu.sync_copy(acc.at[pl.ds(nbase + z * _K, _K)], mbuf)
                pltpu.sync_copy(mbuf, out_hbm.at[g, pl.ds(nbase + z * _K, _K)])
            plsc.subcore_barrier()

    return k(m, dst2, zeros_tile)


def _tc_output(agg, node_feats, node_attrs, W_out0, W_out1, W_sc_r, perm):
    inv_sqrt_d = 1.0 / math.sqrt(_D)
    inv_sqrt_sc = 1.0 / math.sqrt(_D * 4.0)

    def body(agg_ref, nf_ref, na_ref, w0_ref, w1_ref, wsc_ref, pm_ref, o_ref):
        nf = nf_ref[...]
        na = na_ref[...]
        out0 = jnp.dot(agg_ref[0], w0_ref[...], preferred_element_type=jnp.float32) * inv_sqrt_d
        sc_in = jnp.concatenate([nf * na[:, v:v + 1] for v in range(4)], axis=1)
        sc = jnp.dot(sc_in, wsc_ref[...], preferred_element_type=jnp.float32) * inv_sqrt_sc
        s = out0 + sc
        scalars = jax.nn.silu(s[:, :_D])
        gates = jax.nn.silu(s[:, _D:192])
        g0 = jnp.dot(agg_ref[1], w1_ref[...], preferred_element_type=jnp.float32) * inv_sqrt_d * gates
        g1 = jnp.dot(agg_ref[2], w1_ref[...], preferred_element_type=jnp.float32) * inv_sqrt_d * gates
        g2 = jnp.dot(agg_ref[3], w1_ref[...], preferred_element_type=jnp.float32) * inv_sqrt_d * gates
        G = jnp.concatenate([g0, g1, g2], axis=1)
        tail = jnp.dot(G, pm_ref[...], preferred_element_type=jnp.float32)
        o_ref[...] = jnp.concatenate([scalars, tail], axis=1)

    return pl.pallas_call(
        body,
        grid=(10,),
        in_specs=[
            pl.BlockSpec((4, 1000, _D), lambda i: (0, i, 0)),
            pl.BlockSpec((1000, _D), lambda i: (i, 0)),
            pl.BlockSpec((1000, 4), lambda i: (i, 0)),
            pl.BlockSpec((_D, 192), lambda i: (0, 0)),
            pl.BlockSpec((_D, 64), lambda i: (0, 0)),
            pl.BlockSpec((4 * _D, 192), lambda i: (0, 0)),
            pl.BlockSpec((192, 192), lambda i: (0, 0)),
        ],
        out_specs=pl.BlockSpec((1000, 320), lambda i: (i, 0)),
        out_shape=jax.ShapeDtypeStruct((_N, 320), jnp.float32),
    )(agg, node_feats, node_attrs, W_out0, W_out1, W_sc_r, perm)


_PERM_NP = np.zeros((192, 192), dtype=np.float32)
for _c in range(3):
    for _u in range(64):
        _PERM_NP[64 * _c + _u, 3 * _u + _c] = 1.0


def kernel(node_feats, node_attrs, edge_feats, edge_attrs, W_up, W_r1, W_r2, W_out0, W_out1, W_sc, edge_index):
    src = edge_index[0]
    dst = edge_index[1]

    pad = _E_PAD - _E
    src2 = jnp.pad(src, (0, pad)).reshape(_EROWS, _K)
    dst2 = jnp.pad(dst, (0, pad)).reshape(_EROWS, _K)
    edge_feats_p = jnp.pad(edge_feats, ((0, pad), (0, 0)))
    edge_attrs_p = jnp.pad(edge_attrs, ((0, pad), (0, 0)))

    x = _tc_linear_up(node_feats, W_up)
    xs = _sc_gather(x, src2)
    m = _tc_messages(edge_feats_p, edge_attrs_p, xs, W_r1, W_r2)

    zeros_tile = jnp.zeros((_K, _D), jnp.float32)
    agg = _sc_scatter(m, dst2, zeros_tile)

    W_sc_r = W_sc.transpose(1, 0, 2).reshape(4 * _D, 192)
    perm = jnp.asarray(_PERM_NP)
    return _tc_output(agg, node_feats, node_attrs, W_out0, W_out1, W_sc_r, perm)


# speedup vs baseline: 3.1000x; 1.1933x over previous
import functools
import math

import jax
import jax.numpy as jnp
import numpy as np
from jax import lax
from jax.experimental import pallas as pl
from jax.experimental.pallas import tpu as pltpu
from jax.experimental.pallas import tpu_sc as plsc

_N = 10000
_E = 320000
_D = 128
_AVG_NEIGH = 32.0

_NC = 2
_NS = 16
_NW = _NC * _NS
_K = 128

_EROWS = 2560
_E_PAD = _EROWS * _K

_ROWS_PER_W = _EROWS // _NW
_ROWS_PER_T = _EROWS // _NS
_N_PAD = 10240
_NODES_PER_T = _N_PAD // _NS


def _tc_linear_up(node_feats, W_up):
    def body(nf_ref, w_ref, o_ref):
        o_ref[...] = jnp.dot(nf_ref[...], w_ref[...],
                             preferred_element_type=jnp.float32) * (1.0 / math.sqrt(_D))

    return pl.pallas_call(
        body,
        grid=(10,),
        in_specs=[
            pl.BlockSpec((1000, _D), lambda i: (i, 0)),
            pl.BlockSpec((_D, _D), lambda i: (0, 0)),
        ],
        out_specs=pl.BlockSpec((1000, _D), lambda i: (i, 0)),
        out_shape=jax.ShapeDtypeStruct((_N, _D), jnp.float32),
    )(node_feats, W_up)


def _sc_gather(x, src2):
    mesh = plsc.VectorSubcoreMesh(core_axis_name="c", subcore_axis_name="s",
                                  num_cores=_NC, num_subcores=_NS)
    _NSL = 4

    @functools.partial(
        pl.kernel,
        mesh=mesh,
        out_type=jax.ShapeDtypeStruct((_E_PAD, _D), jnp.float32),
        scratch_types=(
            [pltpu.VMEM((_ROWS_PER_W, _K), jnp.int32),
             pltpu.VMEM((_NSL, _K, _D), jnp.float32)]
            + [pltpu.SemaphoreType.DMA] * (2 * _NSL)
        ),
    )
    def k(x_hbm, src_hbm, out_hbm, idx_all, rows, *sems):
        sem_g = sems[:_NSL]
        sem_w = sems[_NSL:]
        wid = lax.axis_index("s") * _NC + lax.axis_index("c")
        base = wid * _ROWS_PER_W

        pltpu.sync_copy(src_hbm.at[pl.ds(base, _ROWS_PER_W)], idx_all)

        def start_gather(t, s):
            pltpu.async_copy(x_hbm.at[idx_all.at[t]], rows.at[s], sem_g[s])

        for s in range(_NSL):
            start_gather(s, s)

        def body(i, carry):
            for s in range(_NSL):
                t = i * _NSL + s
                pltpu.make_async_copy(x_hbm.at[idx_all.at[0]], rows.at[s], sem_g[s]).wait()
                pltpu.async_copy(rows.at[s], out_hbm.at[pl.ds((base + t) * _K, _K)], sem_w[s])
                pltpu.make_async_copy(rows.at[s], out_hbm.at[pl.ds(0, _K)], sem_w[s]).wait()

                @pl.when(t + _NSL < _ROWS_PER_W)
                def _():
                    start_gather(t + _NSL, s)
            return carry

        lax.fori_loop(0, _ROWS_PER_W // _NSL, body, 0)

    return k(x, src2)


def _tc_messages(edge_feats_p, edge_attrs_p, xs, W_r1, W_r2):
    def body(ef_ref, ea_ref, xs_ref, w1_ref, w2_ref, m_ref):
        h = jax.nn.silu(jnp.dot(ef_ref[...], w1_ref[...],
                                preferred_element_type=jnp.float32) * (1.0 / math.sqrt(8.0)))
        w = jnp.dot(h, w2_ref[...], preferred_element_type=jnp.float32) * (1.0 / math.sqrt(64.0))
        wA = w[:, :_D]
        wB = w[:, _D:]
        xs = xs_ref[...]
        scale = 1.0 / _AVG_NEIGH
        m_ref[0] = (wA * xs) * ea_ref[:, 0:1] * scale
        wx = wB * xs
        m_ref[1] = wx * ea_ref[:, 1:2] * scale
        m_ref[2] = wx * ea_ref[:, 2:3] * scale
        m_ref[3] = wx * ea_ref[:, 3:4] * scale

    _BE = 2048
    return pl.pallas_call(
        body,
        grid=(_E_PAD // _BE,),
        in_specs=[
            pl.BlockSpec((_BE, 8), lambda i: (i, 0)),
            pl.BlockSpec((_BE, 4), lambda i: (i, 0)),
            pl.BlockSpec((_BE, _D), lambda i: (i, 0)),
            pl.BlockSpec((8, 64), lambda i: (0, 0)),
            pl.BlockSpec((64, 2 * _D), lambda i: (0, 0)),
        ],
        out_specs=pl.BlockSpec((4, _BE, _D), lambda i: (0, i, 0)),
        out_shape=jax.ShapeDtypeStruct((4, _E_PAD, _D), jnp.float32),
    )(edge_feats_p, edge_attrs_p, xs, W_r1, W_r2)


def _sc_scatter(m, dst2, zeros_tile):
    mesh = plsc.VectorSubcoreMesh(core_axis_name="c", subcore_axis_name="s",
                                  num_cores=_NC, num_subcores=_NS)

    @functools.partial(
        pl.kernel,
        mesh=mesh,
        out_type=jax.ShapeDtypeStruct((4, _N_PAD, _D), jnp.float32),
        scratch_types=(
            [pltpu.VMEM((2, _K), jnp.int32),
             pltpu.VMEM((2, _K, _D), jnp.float32),
             pltpu.VMEM_SHARED((_N_PAD, _D), jnp.float32)]
            + [pltpu.SemaphoreType.DMA] * 4
        ),
    )
    def k(m_hbm, dst_hbm, z_hbm, out_hbm, idx2, mbuf, acc, *sems):
        sem_l = sems[:2]
        sem_s = sems[2:]
        cid = lax.axis_index("c")
        tid = lax.axis_index("s")
        nbase = tid * _NODES_PER_T
        nchunks = _NODES_PER_T // _K

        for gi in range(2):
            g = cid + 2 * gi

            pltpu.sync_copy(z_hbm, mbuf.at[0])
            for z in range(nchunks):
                pltpu.sync_copy(mbuf.at[0], acc.at[pl.ds(nbase + z * _K, _K)])
            plsc.subcore_barrier()

            def start_loads(t, b):
                r = tid * _ROWS_PER_T + t
                pltpu.async_copy(dst_hbm.at[r], idx2.at[b], sem_l[b])
                pltpu.async_copy(m_hbm.at[g, pl.ds(r * _K, _K)], mbuf.at[b], sem_l[b])

            start_loads(0, 0)
            start_loads(1, 1)

            def body(i, carry):
                for b in range(2):
                    t = i * 2 + b
                    pltpu.make_async_copy(dst_hbm.at[0], idx2.at[b], sem_l[b]).wait()
                    pltpu.make_async_copy(m_hbm.at[g, pl.ds(0, _K)], mbuf.at[b], sem_l[b]).wait()
                    pltpu.async_copy(mbuf.at[b], acc.at[idx2.at[b]], sem_s[b], add=True)
                    pltpu.make_async_copy(mbuf.at[b], acc.at[idx2.at[b]], sem_s[b]).wait()

                    @pl.when(t + 2 < _ROWS_PER_T)
                    def _():
                        start_loads(t + 2, b)
                return carry

            lax.fori_loop(0, _ROWS_PER_T // 2, body, 0)
            plsc.subcore_barrier()

            for z in range(nchunks):
                pltpu.sync_copy(acc.at[pl.ds(nbase + z * _K, _K)], mbuf.at[0])
                pltpu.sync_copy(mbuf.at[0], out_hbm.at[g, pl.ds(nbase + z * _K, _K)])
            plsc.subcore_barrier()

    return k(m, dst2, zeros_tile)


def _tc_output(agg, node_feats, node_attrs, W_out0, W_out1, W_sc_r, perm):
    inv_sqrt_d = 1.0 / math.sqrt(_D)
    inv_sqrt_sc = 1.0 / math.sqrt(_D * 4.0)

    def body(agg_ref, nf_ref, na_ref, w0_ref, w1_ref, wsc_ref, pm_ref, o_ref):
        nf = nf_ref[...]
        na = na_ref[...]
        out0 = jnp.dot(agg_ref[0], w0_ref[...], preferred_element_type=jnp.float32) * inv_sqrt_d
        sc_in = jnp.concatenate([nf * na[:, v:v + 1] for v in range(4)], axis=1)
        sc = jnp.dot(sc_in, wsc_ref[...], preferred_element_type=jnp.float32) * inv_sqrt_sc
        s = out0 + sc
        scalars = jax.nn.silu(s[:, :_D])
        gates = jax.nn.silu(s[:, _D:192])
        g0 = jnp.dot(agg_ref[1], w1_ref[...], preferred_element_type=jnp.float32) * inv_sqrt_d * gates
        g1 = jnp.dot(agg_ref[2], w1_ref[...], preferred_element_type=jnp.float32) * inv_sqrt_d * gates
        g2 = jnp.dot(agg_ref[3], w1_ref[...], preferred_element_type=jnp.float32) * inv_sqrt_d * gates
        G = jnp.concatenate([g0, g1, g2], axis=1)
        tail = jnp.dot(G, pm_ref[...], preferred_element_type=jnp.float32)
        o_ref[...] = jnp.concatenate([scalars, tail], axis=1)

    return pl.pallas_call(
        body,
        grid=(10,),
        in_specs=[
            pl.BlockSpec((4, 1000, _D), lambda i: (0, i, 0)),
            pl.BlockSpec((1000, _D), lambda i: (i, 0)),
            pl.BlockSpec((1000, 4), lambda i: (i, 0)),
            pl.BlockSpec((_D, 192), lambda i: (0, 0)),
            pl.BlockSpec((_D, 64), lambda i: (0, 0)),
            pl.BlockSpec((4 * _D, 192), lambda i: (0, 0)),
            pl.BlockSpec((192, 192), lambda i: (0, 0)),
        ],
        out_specs=pl.BlockSpec((1000, 320), lambda i: (i, 0)),
        out_shape=jax.ShapeDtypeStruct((_N, 320), jnp.float32),
    )(agg, node_feats, node_attrs, W_out0, W_out1, W_sc_r, perm)


_PERM_NP = np.zeros((192, 192), dtype=np.float32)
for _c in range(3):
    for _u in range(64):
        _PERM_NP[64 * _c + _u, 3 * _u + _c] = 1.0


def kernel(node_feats, node_attrs, edge_feats, edge_attrs, W_up, W_r1, W_r2, W_out0, W_out1, W_sc, edge_index):
    src = edge_index[0]
    dst = edge_index[1]

    pad = _E_PAD - _E
    src2 = jnp.pad(src, (0, pad)).reshape(_EROWS, _K)
    dst2 = jnp.pad(dst, (0, pad)).reshape(_EROWS, _K)
    edge_feats_p = jnp.pad(edge_feats, ((0, pad), (0, 0)))
    edge_attrs_p = jnp.pad(edge_attrs, ((0, pad), (0, 0)))

    x = _tc_linear_up(node_feats, W_up)
    xs = _sc_gather(x, src2)
    m = _tc_messages(edge_feats_p, edge_attrs_p, xs, W_r1, W_r2)

    zeros_tile = jnp.zeros((_K, _D), jnp.float32)
    agg = _sc_scatter(m, dst2, zeros_tile)

    W_sc_r = W_sc.transpose(1, 0, 2).reshape(4 * _D, 192)
    perm = jnp.asarray(_PERM_NP)
    return _tc_output(agg, node_feats, node_attrs, W_out0, W_out1, W_sc_r, perm)


# speedup vs baseline: 4.1006x; 1.3228x over previous
import functools
import math

import jax
import jax.numpy as jnp
import numpy as np
from jax import lax
from jax.experimental import pallas as pl
from jax.experimental.pallas import tpu as pltpu
from jax.experimental.pallas import tpu_sc as plsc

_N = 10000
_E = 320000
_D = 128
_AVG_NEIGH = 32.0

_NC = 2
_NS = 16
_NW = _NC * _NS
_K = 128

_EROWS = 2560
_E_PAD = _EROWS * _K

_ROWS_PER_W = _EROWS // _NW
_ROWS_PER_T = _EROWS // _NS
_N_PAD = 10240
_NODES_PER_T = _N_PAD // _NS


def _tc_linear_up(node_feats_p, W_up):
    def body(nf_ref, w_ref, o_ref):
        o_ref[...] = jnp.dot(nf_ref[...], w_ref[...],
                             preferred_element_type=jnp.float32) * (1.0 / math.sqrt(_D))

    return pl.pallas_call(
        body,
        grid=(10,),
        in_specs=[
            pl.BlockSpec((1024, _D), lambda i: (i, 0)),
            pl.BlockSpec((_D, _D), lambda i: (0, 0)),
        ],
        out_specs=pl.BlockSpec((1024, _D), lambda i: (i, 0)),
        out_shape=jax.ShapeDtypeStruct((_N_PAD, _D), jnp.float32),
    )(node_feats_p, W_up)


def _sc_gather(x, src2):
    mesh = plsc.VectorSubcoreMesh(core_axis_name="c", subcore_axis_name="s",
                                  num_cores=_NC, num_subcores=_NS)
    _NSL = 2

    @functools.partial(
        pl.kernel,
        mesh=mesh,
        out_type=jax.ShapeDtypeStruct((_E_PAD, _D), jnp.float32),
        scratch_types=(
            [pltpu.VMEM((_ROWS_PER_W, _K), jnp.int32),
             pltpu.VMEM((_NSL, _K, _D), jnp.float32),
             pltpu.VMEM_SHARED((_N_PAD, _D), jnp.float32)]
            + [pltpu.SemaphoreType.DMA] * (2 * _NSL)
        ),
    )
    def k(x_hbm, src_hbm, out_hbm, idx_all, rows, xsh, *sems):
        sem_g = sems[:_NSL]
        sem_w = sems[_NSL:]
        cid = lax.axis_index("c")
        tid = lax.axis_index("s")
        wid = tid * _NC + cid
        base = wid * _ROWS_PER_W

        for z in range(_NODES_PER_T // _K):
            r = tid * _NODES_PER_T + z * _K
            pltpu.sync_copy(x_hbm.at[pl.ds(r, _K)], rows.at[0])
            pltpu.sync_copy(rows.at[0], xsh.at[pl.ds(r, _K)])
        pltpu.sync_copy(src_hbm.at[pl.ds(base, _ROWS_PER_W)], idx_all)
        plsc.subcore_barrier()

        def start_gather(t, s):
            pltpu.async_copy(xsh.at[idx_all.at[t]], rows.at[s], sem_g[s])

        for s in range(_NSL):
            start_gather(s, s)

        def body(i, carry):
            for s in range(_NSL):
                t = i * _NSL + s
                pltpu.make_async_copy(xsh.at[idx_all.at[0]], rows.at[s], sem_g[s]).wait()
                pltpu.async_copy(rows.at[s], out_hbm.at[pl.ds((base + t) * _K, _K)], sem_w[s])
                pltpu.make_async_copy(rows.at[s], out_hbm.at[pl.ds(0, _K)], sem_w[s]).wait()

                @pl.when(t + _NSL < _ROWS_PER_W)
                def _():
                    start_gather(t + _NSL, s)
            return carry

        lax.fori_loop(0, _ROWS_PER_W // _NSL, body, 0)

    return k(x, src2)


def _tc_messages(edge_feats_p, edge_attrs_p, xs, W_r1, W_r2):
    def body(ef_ref, ea_ref, xs_ref, w1_ref, w2_ref, m_ref):
        h = jax.nn.silu(jnp.dot(ef_ref[...], w1_ref[...],
                                preferred_element_type=jnp.float32) * (1.0 / math.sqrt(8.0)))
        w = jnp.dot(h, w2_ref[...], preferred_element_type=jnp.float32) * (1.0 / math.sqrt(64.0))
        wA = w[:, :_D]
        wB = w[:, _D:]
        xs = xs_ref[...]
        scale = 1.0 / _AVG_NEIGH
        m_ref[0] = (wA * xs) * ea_ref[:, 0:1] * scale
        wx = wB * xs
        m_ref[1] = wx * ea_ref[:, 1:2] * scale
        m_ref[2] = wx * ea_ref[:, 2:3] * scale
        m_ref[3] = wx * ea_ref[:, 3:4] * scale

    _BE = 2048
    return pl.pallas_call(
        body,
        grid=(_E_PAD // _BE,),
        in_specs=[
            pl.BlockSpec((_BE, 8), lambda i: (i, 0)),
            pl.BlockSpec((_BE, 4), lambda i: (i, 0)),
            pl.BlockSpec((_BE, _D), lambda i: (i, 0)),
            pl.BlockSpec((8, 64), lambda i: (0, 0)),
            pl.BlockSpec((64, 2 * _D), lambda i: (0, 0)),
        ],
        out_specs=pl.BlockSpec((4, _BE, _D), lambda i: (0, i, 0)),
        out_shape=jax.ShapeDtypeStruct((4, _E_PAD, _D), jnp.float32),
    )(edge_feats_p, edge_attrs_p, xs, W_r1, W_r2)


def _sc_scatter(m, dst2, zeros_tile):
    mesh = plsc.VectorSubcoreMesh(core_axis_name="c", subcore_axis_name="s",
                                  num_cores=_NC, num_subcores=_NS)

    @functools.partial(
        pl.kernel,
        mesh=mesh,
        out_type=jax.ShapeDtypeStruct((4, _N_PAD, _D), jnp.float32),
        scratch_types=(
            [pltpu.VMEM((2, _K), jnp.int32),
             pltpu.VMEM((2, _K, _D), jnp.float32),
             pltpu.VMEM_SHARED((_N_PAD, _D), jnp.float32)]
            + [pltpu.SemaphoreType.DMA] * 4
        ),
    )
    def k(m_hbm, dst_hbm, z_hbm, out_hbm, idx2, mbuf, acc, *sems):
        sem_l = sems[:2]
        sem_s = sems[2:]
        cid = lax.axis_index("c")
        tid = lax.axis_index("s")
        nbase = tid * _NODES_PER_T
        nchunks = _NODES_PER_T // _K

        for gi in range(2):
            g = cid + 2 * gi

            pltpu.sync_copy(z_hbm, mbuf.at[0])
            for z in range(nchunks):
                pltpu.sync_copy(mbuf.at[0], acc.at[pl.ds(nbase + z * _K, _K)])
            plsc.subcore_barrier()

            def start_loads(t, b):
                r = tid * _ROWS_PER_T + t
                pltpu.async_copy(dst_hbm.at[r], idx2.at[b], sem_l[b])
                pltpu.async_copy(m_hbm.at[g, pl.ds(r * _K, _K)], mbuf.at[b], sem_l[b])

            start_loads(0, 0)
            start_loads(1, 1)

            def body(i, carry):
                for b in range(2):
                    t = i * 2 + b
                    pltpu.make_async_copy(dst_hbm.at[0], idx2.at[b], sem_l[b]).wait()
                    pltpu.make_async_copy(m_hbm.at[g, pl.ds(0, _K)], mbuf.at[b], sem_l[b]).wait()
                    pltpu.async_copy(mbuf.at[b], acc.at[idx2.at[b]], sem_s[b], add=True)
                    pltpu.make_async_copy(mbuf.at[b], acc.at[idx2.at[b]], sem_s[b]).wait()

                    @pl.when(t + 2 < _ROWS_PER_T)
                    def _():
                        start_loads(t + 2, b)
                return carry

            lax.fori_loop(0, _ROWS_PER_T // 2, body, 0)
            plsc.subcore_barrier()

            for z in range(nchunks):
                pltpu.sync_copy(acc.at[pl.ds(nbase + z * _K, _K)], mbuf.at[0])
                pltpu.sync_copy(mbuf.at[0], out_hbm.at[g, pl.ds(nbase + z * _K, _K)])
            plsc.subcore_barrier()

    return k(m, dst2, zeros_tile)


def _tc_output(agg, node_feats, node_attrs, W_out0, W_out1, W_sc_r, perm):
    inv_sqrt_d = 1.0 / math.sqrt(_D)
    inv_sqrt_sc = 1.0 / math.sqrt(_D * 4.0)

    def body(agg_ref, nf_ref, na_ref, w0_ref, w1_ref, wsc_ref, pm_ref, o_ref):
        nf = nf_ref[...]
        na = na_ref[...]
        out0 = jnp.dot(agg_ref[0], w0_ref[...], preferred_element_type=jnp.float32) * inv_sqrt_d
        sc_in = jnp.concatenate([nf * na[:, v:v + 1] for v in range(4)], axis=1)
        sc = jnp.dot(sc_in, wsc_ref[...], preferred_element_type=jnp.float32) * inv_sqrt_sc
        s = out0 + sc
        scalars = jax.nn.silu(s[:, :_D])
        gates = jax.nn.silu(s[:, _D:192])
        g0 = jnp.dot(agg_ref[1], w1_ref[...], preferred_element_type=jnp.float32) * inv_sqrt_d * gates
        g1 = jnp.dot(agg_ref[2], w1_ref[...], preferred_element_type=jnp.float32) * inv_sqrt_d * gates
        g2 = jnp.dot(agg_ref[3], w1_ref[...], preferred_element_type=jnp.float32) * inv_sqrt_d * gates
        G = jnp.concatenate([g0, g1, g2], axis=1)
        tail = jnp.dot(G, pm_ref[...], preferred_element_type=jnp.float32)
        o_ref[...] = jnp.concatenate([scalars, tail], axis=1)

    return pl.pallas_call(
        body,
        grid=(10,),
        in_specs=[
            pl.BlockSpec((4, 1000, _D), lambda i: (0, i, 0)),
            pl.BlockSpec((1000, _D), lambda i: (i, 0)),
            pl.BlockSpec((1000, 4), lambda i: (i, 0)),
            pl.BlockSpec((_D, 192), lambda i: (0, 0)),
            pl.BlockSpec((_D, 64), lambda i: (0, 0)),
            pl.BlockSpec((4 * _D, 192), lambda i: (0, 0)),
            pl.BlockSpec((192, 192), lambda i: (0, 0)),
        ],
        out_specs=pl.BlockSpec((1000, 320), lambda i: (i, 0)),
        out_shape=jax.ShapeDtypeStruct((_N, 320), jnp.float32),
    )(agg, node_feats, node_attrs, W_out0, W_out1, W_sc_r, perm)


_PERM_NP = np.zeros((192, 192), dtype=np.float32)
for _c in range(3):
    for _u in range(64):
        _PERM_NP[64 * _c + _u, 3 * _u + _c] = 1.0


def kernel(node_feats, node_attrs, edge_feats, edge_attrs, W_up, W_r1, W_r2, W_out0, W_out1, W_sc, edge_index):
    src = edge_index[0]
    dst = edge_index[1]

    pad = _E_PAD - _E
    src2 = jnp.pad(src, (0, pad)).reshape(_EROWS, _K)
    dst2 = jnp.pad(dst, (0, pad)).reshape(_EROWS, _K)
    edge_feats_p = jnp.pad(edge_feats, ((0, pad), (0, 0)))
    edge_attrs_p = jnp.pad(edge_attrs, ((0, pad), (0, 0)))
    node_feats_p = jnp.pad(node_feats, ((0, _N_PAD - _N), (0, 0)))

    x = _tc_linear_up(node_feats_p, W_up)
    xs = _sc_gather(x, src2)
    m = _tc_messages(edge_feats_p, edge_attrs_p, xs, W_r1, W_r2)

    zeros_tile = jnp.zeros((_K, _D), jnp.float32)
    agg = _sc_scatter(m, dst2, zeros_tile)

    W_sc_r = W_sc.transpose(1, 0, 2).reshape(4 * _D, 192)
    perm = jnp.asarray(_PERM_NP)
    return _tc_output(agg, node_feats, node_attrs, W_out0, W_out1, W_sc_r, perm)
